# Initial kernel scaffold; baseline (speedup 1.0000x reference)
import jax
import jax.numpy as jnp
from jax.experimental import pallas as pl


def kernel(pos, x, batch, c1_W0, c1_b0, c1_g0, c1_be0, c1_W1, c1_b1, c1_g1, c1_be1, c1_W2, c1_b2, c1_g2, c1_be2, c2_W, c2_b, c2_g, c2_be, lin1_W, lin1_b, m_W0, m_b0, m_W1, m_b1, m_W2, m_b2):
    raise NotImplementedError("write your pallas kernel here")


# speedup vs baseline: 13.2637x; 13.2637x over previous
import functools

import jax
import jax.numpy as jnp
from jax import lax
from jax.experimental import pallas as pl
from jax.experimental.pallas import tpu as pltpu
from jax.experimental.pallas import tpu_sc as plsc

N = 8192
B = 8
K = 20
KP = 24
RB = 256
NB = N // RB
CC = 512
E = N * KP
BIG = 1e10
EXBIG = 2e10
IBIG = 2**30

_f32 = jnp.float32


def _dot(a, b):
    return lax.dot_general(a, b, (((1,), (0,)), ((), ())),
                           preferred_element_type=_f32)


def _k1_body(bounds_ref, x0_ref, x0T_ref, b2d_ref, bcol_ref, W0_ref, b0_ref,
             idx_ref, A1_ref, B1_ref, dist_s):
    r = pl.program_id(0)
    x0b = x0_ref[...]
    W0 = W0_ref[...]
    Wa = W0[0:4, :]
    Wb = W0[4:8, :]
    A1_ref[...] = _dot(x0b, Wa - Wb) + b0_ref[...]
    B1_ref[...] = jnp.concatenate(
        [_dot(x0b, Wb), jnp.zeros((RB, 64), _f32)], axis=1)

    d2r = jnp.sum(x0b * x0b, axis=1, keepdims=True)
    rowg = r * RB + lax.broadcasted_iota(jnp.int32, (RB, 1), 0)
    bcol = bcol_ref[...]
    jlo = bounds_ref[r, 0]
    jhi = bounds_ref[r, 1]

    def fill(j, carry):
        coff = pl.multiple_of(j * CC, CC)
        xc = x0T_ref[:, pl.ds(coff, CC)]
        dm = _dot(x0b, xc)
        d2c = jnp.sum(xc * xc, axis=0, keepdims=True)
        dist = d2r + d2c - 2.0 * dm
        colg = coff + lax.broadcasted_iota(jnp.int32, (RB, CC), 1)
        bc = b2d_ref[:, pl.ds(coff, CC)]
        valid = (bc == bcol) & (colg != rowg)
        dist_s[:, pl.ds(coff, CC)] = jnp.where(valid, dist, BIG)
        return carry

    lax.fori_loop(jlo, jhi, fill, 0)

    prev = None
    first = None
    for t in range(K):
        def sel(j, carry, prev=prev, t=t):
            minv, mini = carry
            coff = pl.multiple_of(j * CC, CC)
            chunk = dist_s[:, pl.ds(coff, CC)]
            colg = coff + lax.broadcasted_iota(jnp.int32, (RB, CC), 1)
            if t > 0:
                chunk = jnp.where(colg == prev, EXBIG, chunk)
                dist_s[:, pl.ds(coff, CC)] = chunk
            cmin = jnp.min(chunk, axis=1, keepdims=True)
            cidx = jnp.min(jnp.where(chunk == cmin, colg, IBIG),
                           axis=1, keepdims=True)
            upd = cmin < minv
            return (jnp.where(upd, cmin, minv), jnp.where(upd, cidx, mini))

        minv0 = jnp.full((RB, 1), jnp.inf, _f32)
        mini0 = jnp.zeros((RB, 1), jnp.int32)
        _, mini = lax.fori_loop(jlo, jhi, sel, (minv0, mini0))
        idx_ref[:, pl.ds(t, 1)] = mini
        prev = mini
        if t == 0:
            first = mini
    idx_ref[:, pl.ds(K, KP - K)] = jnp.broadcast_to(first, (RB, KP - K))


def _k1(bounds, x0, x0T, b2d, bcol, W0, b0):
    return pl.pallas_call(
        _k1_body,
        grid=(NB,),
        in_specs=[
            pl.BlockSpec(memory_space=pltpu.SMEM),
            pl.BlockSpec((RB, 4), lambda r: (r, 0)),
            pl.BlockSpec((4, N), lambda r: (0, 0)),
            pl.BlockSpec((1, N), lambda r: (0, 0)),
            pl.BlockSpec((RB, 1), lambda r: (r, 0)),
            pl.BlockSpec((8, 64), lambda r: (0, 0)),
            pl.BlockSpec((1, 64), lambda r: (0, 0)),
        ],
        out_specs=[
            pl.BlockSpec((RB, KP), lambda r: (r, 0)),
            pl.BlockSpec((RB, 64), lambda r: (r, 0)),
            pl.BlockSpec((RB, 128), lambda r: (r, 0)),
        ],
        out_shape=[
            jax.ShapeDtypeStruct((N, KP), jnp.int32),
            jax.ShapeDtypeStruct((N, 64), _f32),
            jax.ShapeDtypeStruct((N, 128), _f32),
        ],
        scratch_shapes=[pltpu.VMEM((RB, N), _f32)],
        compiler_params=pltpu.CompilerParams(
            dimension_semantics=("arbitrary",)),
    )(bounds, x0, x0T, b2d, bcol, W0, b0)


def _sc_gather_fn():
    info = plsc.get_sparse_core_info()
    NC, NS = info.num_cores, info.num_subcores
    NW = NC * NS
    per_w = E // NW
    ROWS_W = per_w // 128
    T = 4
    C = T * 128
    steps = per_w // C
    mesh = plsc.VectorSubcoreMesh(core_axis_name="c", subcore_axis_name="s")

    @functools.partial(
        pl.kernel, mesh=mesh,
        out_type=jax.ShapeDtypeStruct((E, 128), _f32),
        scratch_types=[
            pltpu.VMEM((ROWS_W, 128), jnp.int32),
            pltpu.VMEM((C, 128), _f32),
            pltpu.SemaphoreType.DMA,
        ],
    )
    def k(table, idx2, out, idx_v, rows_v, sem):
        wid = lax.axis_index("s") * NC + lax.axis_index("c")
        pltpu.sync_copy(idx2.at[pl.ds(wid * ROWS_W, ROWS_W)], idx_v)
        base = wid * per_w
        for s in range(steps):
            cps = [pltpu.async_copy(table.at[idx_v.at[s * T + j]],
                                    rows_v.at[pl.ds(j * 128, 128)], sem)
                   for j in range(T)]
            for cp in cps:
                cp.wait()
            pltpu.sync_copy(rows_v, out.at[pl.ds(base + s * C, C)])

    return k


def _sc_gather_max_fn():
    info = plsc.get_sparse_core_info()
    NC, NS = info.num_cores, info.num_subcores
    NW = NC * NS
    rows_w = N // NW
    R = 32
    G = R * KP
    D = 128
    mesh = plsc.VectorSubcoreMesh(core_axis_name="c", subcore_axis_name="s")

    GT = G // 128

    @functools.partial(
        pl.kernel, mesh=mesh,
        out_type=jax.ShapeDtypeStruct((N, D), _f32),
        scratch_types=[
            pltpu.VMEM((rows_w * KP // 128, 128), jnp.int32),
            pltpu.VMEM((G, D), _f32),
            pltpu.VMEM((R, D), _f32),
            pltpu.SemaphoreType.DMA,
        ],
    )
    def k(table, idx2, out, idx_v, rows_v, out_v, sem):
        wid = lax.axis_index("s") * NC + lax.axis_index("c")
        nrows = rows_w * KP // 128
        pltpu.sync_copy(idx2.at[pl.ds(wid * nrows, nrows)], idx_v)
        for c in range(rows_w // R):
            obase = pl.multiple_of(wid * rows_w + c * R, 8)
            cps = [pltpu.async_copy(table.at[idx_v.at[c * GT + j]],
                                    rows_v.at[pl.ds(j * 128, 128)], sem)
                   for j in range(GT)]
            for cp in cps:
                cp.wait()

            def red(rr, carry):
                rbase = rr * KP
                for q in range(D // 16):
                    a = rows_v[rbase, pl.ds(q * 16, 16)]
                    for t in range(1, KP):
                        a = jnp.maximum(a, rows_v[rbase + t, pl.ds(q * 16, 16)])
                    out_v[rr, pl.ds(q * 16, 16)] = a
                return carry

            lax.fori_loop(0, R, red, 0)
            pltpu.sync_copy(out_v, out.at[pl.ds(obase, R)])

    return k


def _k3_body(G1_ref, A1_ref, g0_ref, be0_ref, W1_ref, b1_ref, g1_ref, be1_ref,
             W2_ref, b2_ref, g2_ref, be2_ref, c2W_ref, c2b_ref,
             x1_ref, A2_ref, B2_ref):
    g1r = G1_ref[:, 0:64]
    a1 = A1_ref[...]
    pre = g1r.reshape(RB, KP, 64) + a1.reshape(RB, 1, 64)
    h = jnp.maximum(pre.reshape(RB * KP, 64), 0.0)
    h = g0_ref[...] * h + be0_ref[...]
    h = jnp.maximum(_dot(h, W1_ref[...]) + b1_ref[...], 0.0)
    h = g1_ref[...] * h + be1_ref[...]
    h = jnp.maximum(_dot(h, W2_ref[...]) + b2_ref[...], 0.0)
    h = g2_ref[...] * h + be2_ref[...]
    x1 = jnp.max(h.reshape(RB, KP, 64), axis=1)
    x1_ref[...] = x1
    c2W = c2W_ref[...]
    Wa = c2W[0:64, :]
    Wb = c2W[64:128, :]
    A2_ref[...] = _dot(x1, Wa - Wb) + c2b_ref[...]
    B2_ref[...] = _dot(x1, Wb)


def _k3(G1, A1, g0, be0, W1, b1, g1, be1, W2, b2, g2, be2, c2W, c2b):
    full64 = pl.BlockSpec((1, 64), lambda r: (0, 0))
    return pl.pallas_call(
        _k3_body,
        grid=(NB,),
        in_specs=[
            pl.BlockSpec((RB * KP, 128), lambda r: (r, 0)),
            pl.BlockSpec((RB, 64), lambda r: (r, 0)),
            full64, full64,
            pl.BlockSpec((64, 64), lambda r: (0, 0)), full64,
            full64, full64,
            pl.BlockSpec((64, 64), lambda r: (0, 0)), full64,
            full64, full64,
            pl.BlockSpec((128, 128), lambda r: (0, 0)),
            pl.BlockSpec((1, 128), lambda r: (0, 0)),
        ],
        out_specs=[
            pl.BlockSpec((RB, 64), lambda r: (r, 0)),
            pl.BlockSpec((RB, 128), lambda r: (r, 0)),
            pl.BlockSpec((RB, 128), lambda r: (r, 0)),
        ],
        out_shape=[
            jax.ShapeDtypeStruct((N, 64), _f32),
            jax.ShapeDtypeStruct((N, 128), _f32),
            jax.ShapeDtypeStruct((N, 128), _f32),
        ],
        compiler_params=pltpu.CompilerParams(
            dimension_semantics=("arbitrary",)),
    )(G1, A1, g0, be0, W1, b1, g1, be1, W2, b2, g2, be2, c2W, c2b)


def _k5_body(x1_ref, A2_ref, M2_ref, bcol_ref, c2g_ref, c2be_ref,
             lW_ref, lb_ref, mW0_ref, mb0_ref, mW1_ref, mb1_ref,
             mW2_ref, mb2_ref, out_ref, acc_ref):
    r = pl.program_id(0)

    @pl.when(r == 0)
    def _init():
        acc_ref[...] = jnp.full((B, 1024), -jnp.inf, _f32)

    x2 = c2g_ref[...] * jnp.maximum(A2_ref[...] + M2_ref[...], 0.0) \
        + c2be_ref[...]
    lW = lW_ref[...]
    o1 = _dot(x1_ref[...], lW[0:64, :]) + _dot(x2, lW[64:192, :]) + lb_ref[...]
    bcol = bcol_ref[...]
    for c in range(B):
        m = jnp.where(bcol == c, o1, -jnp.inf)
        mc = jnp.max(m, axis=0, keepdims=True)
        acc_ref[pl.ds(c, 1), :] = jnp.maximum(acc_ref[pl.ds(c, 1), :], mc)

    @pl.when(r == NB - 1)
    def _head():
        o2 = acc_ref[...]
        h = jnp.maximum(_dot(o2, mW0_ref[...]) + mb0_ref[...], 0.0)
        h = jnp.maximum(_dot(h, mW1_ref[...]) + mb1_ref[...], 0.0)
        o = _dot(h, mW2_ref[...]) + mb2_ref[...]
        mx = jnp.max(o, axis=1, keepdims=True)
        lse = jnp.log(jnp.sum(jnp.exp(o - mx), axis=1, keepdims=True))
        out_ref[...] = o - mx - lse


def _k5(x1, A2, M2, bcol, c2g, c2be, lW, lb, mW0, mb0, mW1, mb1, mW2, mb2):
    def full(shape):
        return pl.BlockSpec(shape, lambda r: (0, 0))
    return pl.pallas_call(
        _k5_body,
        grid=(NB,),
        in_specs=[
            pl.BlockSpec((RB, 64), lambda r: (r, 0)),
            pl.BlockSpec((RB, 128), lambda r: (r, 0)),
            pl.BlockSpec((RB, 128), lambda r: (r, 0)),
            pl.BlockSpec((RB, 1), lambda r: (r, 0)),
            full((1, 128)), full((1, 128)),
            full((192, 1024)), full((1, 1024)),
            full((1024, 512)), full((1, 512)),
            full((512, 256)), full((1, 256)),
            full((256, 40)), full((1, 40)),
        ],
        out_specs=pl.BlockSpec((B, 40), lambda r: (0, 0)),
        out_shape=jax.ShapeDtypeStruct((B, 40), _f32),
        scratch_shapes=[pltpu.VMEM((B, 1024), _f32)],
        compiler_params=pltpu.CompilerParams(
            dimension_semantics=("arbitrary",)),
    )(x1, A2, M2, bcol, c2g, c2be, lW, lb, mW0, mb0, mW1, mb1, mW2, mb2)


def kernel(pos, x, batch, c1_W0, c1_b0, c1_g0, c1_be0, c1_W1, c1_b1, c1_g1,
           c1_be1, c1_W2, c1_b2, c1_g2, c1_be2, c2_W, c2_b, c2_g, c2_be,
           lin1_W, lin1_b, m_W0, m_b0, m_W1, m_b1, m_W2, m_b2):
    x0 = jnp.concatenate([pos, x], axis=1)
    x0T = x0.T
    batch = batch.astype(jnp.int32)
    starts = jnp.searchsorted(batch, jnp.arange(B + 1, dtype=jnp.int32),
                              side="left").astype(jnp.int32)
    rb = jnp.arange(NB, dtype=jnp.int32) * RB
    cf = batch[rb]
    cl = batch[rb + RB - 1]
    jlo = starts[cf] // CC
    jhi = (starts[cl + 1] + CC - 1) // CC
    bounds = jnp.stack([jlo, jhi], axis=1).astype(jnp.int32)

    row = lambda v: v.reshape(1, -1)
    idx, A1, B1 = _k1(bounds, x0, x0T, batch.reshape(1, N),
                      batch.reshape(N, 1), c1_W0, row(c1_b0))
    idx2 = idx.reshape(E // 128, 128)
    G1 = _sc_gather_fn()(B1, idx2)
    x1, A2, B2 = _k3(G1, A1, row(c1_g0), row(c1_be0), c1_W1, row(c1_b1),
                     row(c1_g1), row(c1_be1), c1_W2, row(c1_b2),
                     row(c1_g2), row(c1_be2), c2_W, row(c2_b))
    M2 = _sc_gather_max_fn()(B2, idx2)
    return _k5(x1, A2, M2, batch.reshape(N, 1), row(c2_g), row(c2_be),
               lin1_W, row(lin1_b), m_W0, row(m_b0), m_W1, row(m_b1),
               m_W2, row(m_b2))


# speedup vs baseline: 21.1045x; 1.5911x over previous
import functools

import jax
import jax.numpy as jnp
from jax import lax
from jax.experimental import pallas as pl
from jax.experimental.pallas import tpu as pltpu
from jax.experimental.pallas import tpu_sc as plsc

N = 8192
B = 8
K = 20
KP = 24
RB = 256
NB = N // RB
CC = 512
E = N * KP
BIG = 1e10
EXBIG = 2e10
IBIG = 2**30

_f32 = jnp.float32


def _dot(a, b):
    return lax.dot_general(a, b, (((1,), (0,)), ((), ())),
                           preferred_element_type=_f32)


def _k1_body(bounds_ref, x0_ref, x0T_ref, b2d_ref, bcol_ref, W0_ref, b0_ref,
             idx_ref, A1_ref, B1_ref, key_s):
    r = pl.program_id(0)
    x0b = x0_ref[...]
    W0 = W0_ref[...]
    Wa = W0[0:4, :]
    Wb = W0[4:8, :]
    A1_ref[...] = _dot(x0b, Wa - Wb) + b0_ref[...]
    B1_ref[...] = jnp.concatenate(
        [_dot(x0b, Wb), jnp.zeros((RB, 64), _f32)], axis=1)

    d2r = jnp.sum(x0b * x0b, axis=1, keepdims=True)
    rowg = r * RB + lax.broadcasted_iota(jnp.int32, (RB, 1), 0)
    bcol = bcol_ref[...]
    jlo = bounds_ref[r, 0]
    jhi = bounds_ref[r, 1]

    def fill(j, carry):
        coff = pl.multiple_of(j * CC, CC)
        xc = x0T_ref[:, pl.ds(coff, CC)]
        dm = _dot(x0b, xc)
        d2c = jnp.sum(xc * xc, axis=0, keepdims=True)
        dist = jnp.maximum(d2r + d2c - 2.0 * dm, 0.0)
        colg = coff + lax.broadcasted_iota(jnp.int32, (RB, CC), 1)
        bc = b2d_ref[:, pl.ds(coff, CC)]
        valid = (bc == bcol) & (colg != rowg)
        bits = lax.bitcast_convert_type(jnp.where(valid, dist, BIG), jnp.int32)
        key_s[:, pl.ds(coff, CC)] = (bits & jnp.int32(-8192)) | colg
        return carry

    lax.fori_loop(jlo, jhi, fill, 0)

    MAXI = 0x7FFFFFFF
    prev = None
    first = None
    for t in range(K):
        def sel(j, macc, prev=prev, t=t):
            coff = pl.multiple_of(j * CC, CC)
            kc = key_s[:, pl.ds(coff, CC)]
            if t > 0:
                kc = jnp.where(kc == prev, MAXI, kc)
                key_s[:, pl.ds(coff, CC)] = kc
            fold = jnp.minimum(
                jnp.minimum(kc[:, 0:128], kc[:, 128:256]),
                jnp.minimum(kc[:, 256:384], kc[:, 384:512]))
            return jnp.minimum(macc, fold)

        macc0 = jnp.full((RB, 128), MAXI, jnp.int32)
        macc = lax.fori_loop(jlo, jhi, sel, macc0)
        mk = jnp.min(macc, axis=1, keepdims=True)
        col = mk & 0x1FFF
        idx_ref[:, pl.ds(t, 1)] = col
        prev = mk
        if t == 0:
            first = col
    idx_ref[:, pl.ds(K, KP - K)] = jnp.broadcast_to(first, (RB, KP - K))


def _k1(bounds, x0, x0T, b2d, bcol, W0, b0):
    return pl.pallas_call(
        _k1_body,
        grid=(NB,),
        in_specs=[
            pl.BlockSpec(memory_space=pltpu.SMEM),
            pl.BlockSpec((RB, 4), lambda r: (r, 0)),
            pl.BlockSpec((4, N), lambda r: (0, 0)),
            pl.BlockSpec((1, N), lambda r: (0, 0)),
            pl.BlockSpec((RB, 1), lambda r: (r, 0)),
            pl.BlockSpec((8, 64), lambda r: (0, 0)),
            pl.BlockSpec((1, 64), lambda r: (0, 0)),
        ],
        out_specs=[
            pl.BlockSpec((RB, KP), lambda r: (r, 0)),
            pl.BlockSpec((RB, 64), lambda r: (r, 0)),
            pl.BlockSpec((RB, 128), lambda r: (r, 0)),
        ],
        out_shape=[
            jax.ShapeDtypeStruct((N, KP), jnp.int32),
            jax.ShapeDtypeStruct((N, 64), _f32),
            jax.ShapeDtypeStruct((N, 128), _f32),
        ],
        scratch_shapes=[pltpu.VMEM((RB, N), jnp.int32)],
        compiler_params=pltpu.CompilerParams(
            dimension_semantics=("arbitrary",)),
    )(bounds, x0, x0T, b2d, bcol, W0, b0)


def _sc_gather_fn():
    info = plsc.get_sparse_core_info()
    NC, NS = info.num_cores, info.num_subcores
    NW = NC * NS
    per_w = E // NW
    ROWS_W = per_w // 128
    T = 4
    C = T * 128
    steps = per_w // C
    mesh = plsc.VectorSubcoreMesh(core_axis_name="c", subcore_axis_name="s")

    @functools.partial(
        pl.kernel, mesh=mesh,
        out_type=jax.ShapeDtypeStruct((E, 128), _f32),
        scratch_types=[
            pltpu.VMEM((ROWS_W, 128), jnp.int32),
            pltpu.VMEM((C, 128), _f32),
            pltpu.SemaphoreType.DMA,
        ],
    )
    def k(table, idx2, out, idx_v, rows_v, sem):
        wid = lax.axis_index("s") * NC + lax.axis_index("c")
        pltpu.sync_copy(idx2.at[pl.ds(wid * ROWS_W, ROWS_W)], idx_v)
        base = wid * per_w
        for s in range(steps):
            cps = [pltpu.async_copy(table.at[idx_v.at[s * T + j]],
                                    rows_v.at[pl.ds(j * 128, 128)], sem)
                   for j in range(T)]
            for cp in cps:
                cp.wait()
            pltpu.sync_copy(rows_v, out.at[pl.ds(base + s * C, C)])

    return k


def _sc_gather_max_fn():
    info = plsc.get_sparse_core_info()
    NC, NS = info.num_cores, info.num_subcores
    NW = NC * NS
    rows_w = N // NW
    R = 32
    G = R * KP
    D = 128
    mesh = plsc.VectorSubcoreMesh(core_axis_name="c", subcore_axis_name="s")

    GT = G // 128

    @functools.partial(
        pl.kernel, mesh=mesh,
        out_type=jax.ShapeDtypeStruct((N, D), _f32),
        scratch_types=[
            pltpu.VMEM((rows_w * KP // 128, 128), jnp.int32),
            pltpu.VMEM((G, D), _f32),
            pltpu.VMEM((R, D), _f32),
            pltpu.SemaphoreType.DMA,
        ],
    )
    def k(table, idx2, out, idx_v, rows_v, out_v, sem):
        wid = lax.axis_index("s") * NC + lax.axis_index("c")
        nrows = rows_w * KP // 128
        pltpu.sync_copy(idx2.at[pl.ds(wid * nrows, nrows)], idx_v)
        for c in range(rows_w // R):
            obase = pl.multiple_of(wid * rows_w + c * R, 8)
            cps = [pltpu.async_copy(table.at[idx_v.at[c * GT + j]],
                                    rows_v.at[pl.ds(j * 128, 128)], sem)
                   for j in range(GT)]
            for cp in cps:
                cp.wait()

            def red(rr, carry):
                rbase = rr * KP
                for q in range(D // 16):
                    a = rows_v[rbase, pl.ds(q * 16, 16)]
                    for t in range(1, KP):
                        a = jnp.maximum(a, rows_v[rbase + t, pl.ds(q * 16, 16)])
                    out_v[rr, pl.ds(q * 16, 16)] = a
                return carry

            lax.fori_loop(0, R, red, 0)
            pltpu.sync_copy(out_v, out.at[pl.ds(obase, R)])

    return k


def _k3_body(G1_ref, A1_ref, g0_ref, be0_ref, W1_ref, b1_ref, g1_ref, be1_ref,
             W2_ref, b2_ref, g2_ref, be2_ref, c2W_ref, c2b_ref,
             x1_ref, A2_ref, B2_ref):
    g1r = G1_ref[:, 0:64]
    a1 = A1_ref[...]
    pre = g1r.reshape(RB, KP, 64) + a1.reshape(RB, 1, 64)
    h = jnp.maximum(pre.reshape(RB * KP, 64), 0.0)
    h = g0_ref[...] * h + be0_ref[...]
    h = jnp.maximum(_dot(h, W1_ref[...]) + b1_ref[...], 0.0)
    h = g1_ref[...] * h + be1_ref[...]
    h = jnp.maximum(_dot(h, W2_ref[...]) + b2_ref[...], 0.0)
    h = g2_ref[...] * h + be2_ref[...]
    x1 = jnp.max(h.reshape(RB, KP, 64), axis=1)
    x1_ref[...] = x1
    c2W = c2W_ref[...]
    Wa = c2W[0:64, :]
    Wb = c2W[64:128, :]
    A2_ref[...] = _dot(x1, Wa - Wb) + c2b_ref[...]
    B2_ref[...] = _dot(x1, Wb)


def _k3(G1, A1, g0, be0, W1, b1, g1, be1, W2, b2, g2, be2, c2W, c2b):
    full64 = pl.BlockSpec((1, 64), lambda r: (0, 0))
    return pl.pallas_call(
        _k3_body,
        grid=(NB,),
        in_specs=[
            pl.BlockSpec((RB * KP, 128), lambda r: (r, 0)),
            pl.BlockSpec((RB, 64), lambda r: (r, 0)),
            full64, full64,
            pl.BlockSpec((64, 64), lambda r: (0, 0)), full64,
            full64, full64,
            pl.BlockSpec((64, 64), lambda r: (0, 0)), full64,
            full64, full64,
            pl.BlockSpec((128, 128), lambda r: (0, 0)),
            pl.BlockSpec((1, 128), lambda r: (0, 0)),
        ],
        out_specs=[
            pl.BlockSpec((RB, 64), lambda r: (r, 0)),
            pl.BlockSpec((RB, 128), lambda r: (r, 0)),
            pl.BlockSpec((RB, 128), lambda r: (r, 0)),
        ],
        out_shape=[
            jax.ShapeDtypeStruct((N, 64), _f32),
            jax.ShapeDtypeStruct((N, 128), _f32),
            jax.ShapeDtypeStruct((N, 128), _f32),
        ],
        compiler_params=pltpu.CompilerParams(
            dimension_semantics=("arbitrary",)),
    )(G1, A1, g0, be0, W1, b1, g1, be1, W2, b2, g2, be2, c2W, c2b)


def _k5_body(x1_ref, A2_ref, M2_ref, bcol_ref, c2g_ref, c2be_ref,
             lW_ref, lb_ref, mW0_ref, mb0_ref, mW1_ref, mb1_ref,
             mW2_ref, mb2_ref, out_ref, acc_ref):
    r = pl.program_id(0)

    @pl.when(r == 0)
    def _init():
        acc_ref[...] = jnp.full((B, 1024), -jnp.inf, _f32)

    x2 = c2g_ref[...] * jnp.maximum(A2_ref[...] + M2_ref[...], 0.0) \
        + c2be_ref[...]
    lW = lW_ref[...]
    o1 = _dot(x1_ref[...], lW[0:64, :]) + _dot(x2, lW[64:192, :]) + lb_ref[...]
    bcol = bcol_ref[...]
    for c in range(B):
        m = jnp.where(bcol == c, o1, -jnp.inf)
        mc = jnp.max(m, axis=0, keepdims=True)
        acc_ref[pl.ds(c, 1), :] = jnp.maximum(acc_ref[pl.ds(c, 1), :], mc)

    @pl.when(r == NB - 1)
    def _head():
        o2 = acc_ref[...]
        h = jnp.maximum(_dot(o2, mW0_ref[...]) + mb0_ref[...], 0.0)
        h = jnp.maximum(_dot(h, mW1_ref[...]) + mb1_ref[...], 0.0)
        o = _dot(h, mW2_ref[...]) + mb2_ref[...]
        mx = jnp.max(o, axis=1, keepdims=True)
        lse = jnp.log(jnp.sum(jnp.exp(o - mx), axis=1, keepdims=True))
        out_ref[...] = o - mx - lse


def _k5(x1, A2, M2, bcol, c2g, c2be, lW, lb, mW0, mb0, mW1, mb1, mW2, mb2):
    def full(shape):
        return pl.BlockSpec(shape, lambda r: (0, 0))
    return pl.pallas_call(
        _k5_body,
        grid=(NB,),
        in_specs=[
            pl.BlockSpec((RB, 64), lambda r: (r, 0)),
            pl.BlockSpec((RB, 128), lambda r: (r, 0)),
            pl.BlockSpec((RB, 128), lambda r: (r, 0)),
            pl.BlockSpec((RB, 1), lambda r: (r, 0)),
            full((1, 128)), full((1, 128)),
            full((192, 1024)), full((1, 1024)),
            full((1024, 512)), full((1, 512)),
            full((512, 256)), full((1, 256)),
            full((256, 40)), full((1, 40)),
        ],
        out_specs=pl.BlockSpec((B, 40), lambda r: (0, 0)),
        out_shape=jax.ShapeDtypeStruct((B, 40), _f32),
        scratch_shapes=[pltpu.VMEM((B, 1024), _f32)],
        compiler_params=pltpu.CompilerParams(
            dimension_semantics=("arbitrary",)),
    )(x1, A2, M2, bcol, c2g, c2be, lW, lb, mW0, mb0, mW1, mb1, mW2, mb2)


def kernel(pos, x, batch, c1_W0, c1_b0, c1_g0, c1_be0, c1_W1, c1_b1, c1_g1,
           c1_be1, c1_W2, c1_b2, c1_g2, c1_be2, c2_W, c2_b, c2_g, c2_be,
           lin1_W, lin1_b, m_W0, m_b0, m_W1, m_b1, m_W2, m_b2):
    x0 = jnp.concatenate([pos, x], axis=1)
    x0T = x0.T
    batch = batch.astype(jnp.int32)
    starts = jnp.searchsorted(batch, jnp.arange(B + 1, dtype=jnp.int32),
                              side="left").astype(jnp.int32)
    rb = jnp.arange(NB, dtype=jnp.int32) * RB
    cf = batch[rb]
    cl = batch[rb + RB - 1]
    jlo = starts[cf] // CC
    jhi = (starts[cl + 1] + CC - 1) // CC
    bounds = jnp.stack([jlo, jhi], axis=1).astype(jnp.int32)

    row = lambda v: v.reshape(1, -1)
    idx, A1, B1 = _k1(bounds, x0, x0T, batch.reshape(1, N),
                      batch.reshape(N, 1), c1_W0, row(c1_b0))
    idx2 = idx.reshape(E // 128, 128)
    G1 = _sc_gather_fn()(B1, idx2)
    x1, A2, B2 = _k3(G1, A1, row(c1_g0), row(c1_be0), c1_W1, row(c1_b1),
                     row(c1_g1), row(c1_be1), c1_W2, row(c1_b2),
                     row(c1_g2), row(c1_be2), c2_W, row(c2_b))
    M2 = _sc_gather_max_fn()(B2, idx2)
    return _k5(x1, A2, M2, batch.reshape(N, 1), row(c2_g), row(c2_be),
               lin1_W, row(lin1_b), m_W0, row(m_b0), m_W1, row(m_b1),
               m_W2, row(m_b2))


# speedup vs baseline: 26.3084x; 1.2466x over previous
import functools

import jax
import jax.numpy as jnp
from jax import lax
from jax.experimental import pallas as pl
from jax.experimental.pallas import tpu as pltpu
from jax.experimental.pallas import tpu_sc as plsc

N = 8192
B = 8
K = 20
KP = 24
RB = 256
NB = N // RB
CC = 512
E = N * KP
BIG = 1e10
EXBIG = 2e10
IBIG = 2**30

_f32 = jnp.float32


def _dot(a, b):
    return lax.dot_general(a, b, (((1,), (0,)), ((), ())),
                           preferred_element_type=_f32)


def _k1_body(bounds_ref, x0_ref, x0f_ref, x0T_ref, rlo_ref, rhi_ref,
             W0_ref, b0_ref, idxT_ref, A1_ref, B1_ref, key_s):
    r = pl.program_id(0)
    x0b = x0_ref[...]
    W0 = W0_ref[...]
    Wa = W0[0:4, :]
    Wb = W0[4:8, :]
    A1_ref[...] = _dot(x0b, Wa - Wb) + b0_ref[...]
    B1_ref[...] = jnp.concatenate(
        [_dot(x0b, Wb), jnp.zeros((RB, 64), _f32)], axis=1)

    xrT = x0T_ref[...]
    d2r = jnp.sum(xrT * xrT, axis=0, keepdims=True)
    rowg = r * RB + lax.broadcasted_iota(jnp.int32, (1, RB), 1)
    rlo = rlo_ref[...]
    rhi = rhi_ref[...]
    jlo = bounds_ref[r, 0]
    jhi = bounds_ref[r, 1]

    def fill(j, carry):
        coff = pl.multiple_of(j * CC, CC)
        xc = x0f_ref[pl.ds(coff, CC), :]
        dm = _dot(xc, xrT)
        d2c = jnp.sum(xc * xc, axis=1, keepdims=True)
        dist = jnp.maximum(d2c + d2r - 2.0 * dm, 0.0)
        colg = coff + lax.broadcasted_iota(jnp.int32, (CC, 1), 0)
        valid = (colg >= rlo) & (colg < rhi) & (colg != rowg)
        bits = lax.bitcast_convert_type(jnp.where(valid, dist, BIG), jnp.int32)
        key_s[pl.ds(coff, CC), :] = (bits & jnp.int32(-8192)) | colg
        return carry

    lax.fori_loop(jlo, jhi, fill, 0)

    MAXI = 0x7FFFFFFF
    prev = None
    first = None
    for t in range(K):
        def sel(j, macc, prev=prev, t=t):
            coff = pl.multiple_of(j * CC, CC)
            kc = key_s[pl.ds(coff, CC), :]
            if t > 0:
                kc = jnp.where(kc == prev, MAXI, kc)
                key_s[pl.ds(coff, CC), :] = kc
            fold = jnp.min(kc.reshape(CC // 8, 8, RB), axis=0)
            return jnp.minimum(macc, fold)

        macc0 = jnp.full((8, RB), MAXI, jnp.int32)
        macc = lax.fori_loop(jlo, jhi, sel, macc0)
        mk = jnp.min(macc, axis=0, keepdims=True)
        col = mk & 0x1FFF
        idxT_ref[pl.ds(t, 1), :] = col
        prev = mk
        if t == 0:
            first = col
    idxT_ref[pl.ds(K, KP - K), :] = jnp.broadcast_to(first, (KP - K, RB))


def _k1(bounds, x0, x0T, rlo, rhi, W0, b0):
    return pl.pallas_call(
        _k1_body,
        grid=(NB,),
        in_specs=[
            pl.BlockSpec(memory_space=pltpu.SMEM),
            pl.BlockSpec((RB, 4), lambda r: (r, 0)),
            pl.BlockSpec((N, 4), lambda r: (0, 0)),
            pl.BlockSpec((4, RB), lambda r: (0, r)),
            pl.BlockSpec((1, RB), lambda r: (0, r)),
            pl.BlockSpec((1, RB), lambda r: (0, r)),
            pl.BlockSpec((8, 64), lambda r: (0, 0)),
            pl.BlockSpec((1, 64), lambda r: (0, 0)),
        ],
        out_specs=[
            pl.BlockSpec((KP, RB), lambda r: (0, r)),
            pl.BlockSpec((RB, 64), lambda r: (r, 0)),
            pl.BlockSpec((RB, 128), lambda r: (r, 0)),
        ],
        out_shape=[
            jax.ShapeDtypeStruct((KP, N), jnp.int32),
            jax.ShapeDtypeStruct((N, 64), _f32),
            jax.ShapeDtypeStruct((N, 128), _f32),
        ],
        scratch_shapes=[pltpu.VMEM((N, RB), jnp.int32)],
        compiler_params=pltpu.CompilerParams(
            dimension_semantics=("arbitrary",)),
    )(bounds, x0, x0, x0T, rlo, rhi, W0, b0)


def _sc_gather_fn():
    info = plsc.get_sparse_core_info()
    NC, NS = info.num_cores, info.num_subcores
    NW = NC * NS
    per_w = E // NW
    ROWS_W = per_w // 128
    T = 4
    C = T * 128
    steps = per_w // C
    mesh = plsc.VectorSubcoreMesh(core_axis_name="c", subcore_axis_name="s")

    @functools.partial(
        pl.kernel, mesh=mesh,
        out_type=jax.ShapeDtypeStruct((E, 128), _f32),
        scratch_types=[
            pltpu.VMEM((ROWS_W, 128), jnp.int32),
            pltpu.VMEM((C, 128), _f32),
            pltpu.SemaphoreType.DMA,
        ],
    )
    def k(table, idx2, out, idx_v, rows_v, sem):
        wid = lax.axis_index("s") * NC + lax.axis_index("c")
        pltpu.sync_copy(idx2.at[pl.ds(wid * ROWS_W, ROWS_W)], idx_v)
        base = wid * per_w
        for s in range(steps):
            cps = [pltpu.async_copy(table.at[idx_v.at[s * T + j]],
                                    rows_v.at[pl.ds(j * 128, 128)], sem)
                   for j in range(T)]
            for cp in cps:
                cp.wait()
            pltpu.sync_copy(rows_v, out.at[pl.ds(base + s * C, C)])

    return k


def _sc_gather_max_fn():
    info = plsc.get_sparse_core_info()
    NC, NS = info.num_cores, info.num_subcores
    NW = NC * NS
    rows_w = N // NW
    R = 32
    G = R * KP
    D = 128
    mesh = plsc.VectorSubcoreMesh(core_axis_name="c", subcore_axis_name="s")

    GT = G // 128

    @functools.partial(
        pl.kernel, mesh=mesh,
        out_type=jax.ShapeDtypeStruct((N, D), _f32),
        scratch_types=[
            pltpu.VMEM((rows_w * KP // 128, 128), jnp.int32),
            pltpu.VMEM((G, D), _f32),
            pltpu.VMEM((R, D), _f32),
            pltpu.SemaphoreType.DMA,
        ],
    )
    def k(table, idx2, out, idx_v, rows_v, out_v, sem):
        wid = lax.axis_index("s") * NC + lax.axis_index("c")
        nrows = rows_w * KP // 128
        pltpu.sync_copy(idx2.at[pl.ds(wid * nrows, nrows)], idx_v)
        for c in range(rows_w // R):
            obase = pl.multiple_of(wid * rows_w + c * R, 8)
            cps = [pltpu.async_copy(table.at[idx_v.at[c * GT + j]],
                                    rows_v.at[pl.ds(j * 128, 128)], sem)
                   for j in range(GT)]
            for cp in cps:
                cp.wait()

            def red(rr, carry):
                rbase = rr * KP
                for q in range(D // 16):
                    a = rows_v[rbase, pl.ds(q * 16, 16)]
                    for t in range(1, KP):
                        a = jnp.maximum(a, rows_v[rbase + t, pl.ds(q * 16, 16)])
                    out_v[rr, pl.ds(q * 16, 16)] = a
                return carry

            lax.fori_loop(0, R, red, 0)
            pltpu.sync_copy(out_v, out.at[pl.ds(obase, R)])

    return k


def _k3_body(G1_ref, A1_ref, g0_ref, be0_ref, W1_ref, b1_ref, g1_ref, be1_ref,
             W2_ref, b2_ref, g2_ref, be2_ref, c2W_ref, c2b_ref,
             x1_ref, A2_ref, B2_ref):
    g1r = G1_ref[:, 0:64]
    a1 = A1_ref[...]
    pre = g1r.reshape(RB, KP, 64) + a1.reshape(RB, 1, 64)
    h = jnp.maximum(pre.reshape(RB * KP, 64), 0.0)
    h = g0_ref[...] * h + be0_ref[...]
    h = jnp.maximum(_dot(h, W1_ref[...]) + b1_ref[...], 0.0)
    h = g1_ref[...] * h + be1_ref[...]
    h = jnp.maximum(_dot(h, W2_ref[...]) + b2_ref[...], 0.0)
    h = g2_ref[...] * h + be2_ref[...]
    x1 = jnp.max(h.reshape(RB, KP, 64), axis=1)
    x1_ref[...] = x1
    c2W = c2W_ref[...]
    Wa = c2W[0:64, :]
    Wb = c2W[64:128, :]
    A2_ref[...] = _dot(x1, Wa - Wb) + c2b_ref[...]
    B2_ref[...] = _dot(x1, Wb)


def _k3(G1, A1, g0, be0, W1, b1, g1, be1, W2, b2, g2, be2, c2W, c2b):
    full64 = pl.BlockSpec((1, 64), lambda r: (0, 0))
    return pl.pallas_call(
        _k3_body,
        grid=(NB,),
        in_specs=[
            pl.BlockSpec((RB * KP, 128), lambda r: (r, 0)),
            pl.BlockSpec((RB, 64), lambda r: (r, 0)),
            full64, full64,
            pl.BlockSpec((64, 64), lambda r: (0, 0)), full64,
            full64, full64,
            pl.BlockSpec((64, 64), lambda r: (0, 0)), full64,
            full64, full64,
            pl.BlockSpec((128, 128), lambda r: (0, 0)),
            pl.BlockSpec((1, 128), lambda r: (0, 0)),
        ],
        out_specs=[
            pl.BlockSpec((RB, 64), lambda r: (r, 0)),
            pl.BlockSpec((RB, 128), lambda r: (r, 0)),
            pl.BlockSpec((RB, 128), lambda r: (r, 0)),
        ],
        out_shape=[
            jax.ShapeDtypeStruct((N, 64), _f32),
            jax.ShapeDtypeStruct((N, 128), _f32),
            jax.ShapeDtypeStruct((N, 128), _f32),
        ],
        compiler_params=pltpu.CompilerParams(
            dimension_semantics=("arbitrary",)),
    )(G1, A1, g0, be0, W1, b1, g1, be1, W2, b2, g2, be2, c2W, c2b)


def _k5_body(x1_ref, A2_ref, M2_ref, bcol_ref, c2g_ref, c2be_ref,
             lW_ref, lb_ref, mW0_ref, mb0_ref, mW1_ref, mb1_ref,
             mW2_ref, mb2_ref, out_ref, acc_ref):
    r = pl.program_id(0)

    @pl.when(r == 0)
    def _init():
        acc_ref[...] = jnp.full((B, 1024), -jnp.inf, _f32)

    x2 = c2g_ref[...] * jnp.maximum(A2_ref[...] + M2_ref[...], 0.0) \
        + c2be_ref[...]
    lW = lW_ref[...]
    o1 = _dot(x1_ref[...], lW[0:64, :]) + _dot(x2, lW[64:192, :]) + lb_ref[...]
    bcol = bcol_ref[...]
    for c in range(B):
        m = jnp.where(bcol == c, o1, -jnp.inf)
        mc = jnp.max(m, axis=0, keepdims=True)
        acc_ref[pl.ds(c, 1), :] = jnp.maximum(acc_ref[pl.ds(c, 1), :], mc)

    @pl.when(r == NB - 1)
    def _head():
        o2 = acc_ref[...]
        h = jnp.maximum(_dot(o2, mW0_ref[...]) + mb0_ref[...], 0.0)
        h = jnp.maximum(_dot(h, mW1_ref[...]) + mb1_ref[...], 0.0)
        o = _dot(h, mW2_ref[...]) + mb2_ref[...]
        mx = jnp.max(o, axis=1, keepdims=True)
        lse = jnp.log(jnp.sum(jnp.exp(o - mx), axis=1, keepdims=True))
        out_ref[...] = o - mx - lse


def _k5(x1, A2, M2, bcol, c2g, c2be, lW, lb, mW0, mb0, mW1, mb1, mW2, mb2):
    def full(shape):
        return pl.BlockSpec(shape, lambda r: (0, 0))
    return pl.pallas_call(
        _k5_body,
        grid=(NB,),
        in_specs=[
            pl.BlockSpec((RB, 64), lambda r: (r, 0)),
            pl.BlockSpec((RB, 128), lambda r: (r, 0)),
            pl.BlockSpec((RB, 128), lambda r: (r, 0)),
            pl.BlockSpec((RB, 1), lambda r: (r, 0)),
            full((1, 128)), full((1, 128)),
            full((192, 1024)), full((1, 1024)),
            full((1024, 512)), full((1, 512)),
            full((512, 256)), full((1, 256)),
            full((256, 40)), full((1, 40)),
        ],
        out_specs=pl.BlockSpec((B, 40), lambda r: (0, 0)),
        out_shape=jax.ShapeDtypeStruct((B, 40), _f32),
        scratch_shapes=[pltpu.VMEM((B, 1024), _f32)],
        compiler_params=pltpu.CompilerParams(
            dimension_semantics=("arbitrary",)),
    )(x1, A2, M2, bcol, c2g, c2be, lW, lb, mW0, mb0, mW1, mb1, mW2, mb2)


def kernel(pos, x, batch, c1_W0, c1_b0, c1_g0, c1_be0, c1_W1, c1_b1, c1_g1,
           c1_be1, c1_W2, c1_b2, c1_g2, c1_be2, c2_W, c2_b, c2_g, c2_be,
           lin1_W, lin1_b, m_W0, m_b0, m_W1, m_b1, m_W2, m_b2):
    x0 = jnp.concatenate([pos, x], axis=1)
    x0T = x0.T
    batch = batch.astype(jnp.int32)
    starts = jnp.searchsorted(batch, jnp.arange(B + 1, dtype=jnp.int32),
                              side="left").astype(jnp.int32)
    rb = jnp.arange(NB, dtype=jnp.int32) * RB
    cf = batch[rb]
    cl = batch[rb + RB - 1]
    jlo = starts[cf] // CC
    jhi = (starts[cl + 1] + CC - 1) // CC
    bounds = jnp.stack([jlo, jhi], axis=1).astype(jnp.int32)

    row = lambda v: v.reshape(1, -1)
    rlo = starts[batch].reshape(1, N)
    rhi = starts[batch + 1].reshape(1, N)
    idxT, A1, B1 = _k1(bounds, x0, x0T, rlo, rhi, c1_W0, row(c1_b0))
    idx2 = idxT.T.reshape(E // 128, 128)
    G1 = _sc_gather_fn()(B1, idx2)
    x1, A2, B2 = _k3(G1, A1, row(c1_g0), row(c1_be0), c1_W1, row(c1_b1),
                     row(c1_g1), row(c1_be1), c1_W2, row(c1_b2),
                     row(c1_g2), row(c1_be2), c2_W, row(c2_b))
    M2 = _sc_gather_max_fn()(B2, idx2)
    return _k5(x1, A2, M2, batch.reshape(N, 1), row(c2_g), row(c2_be),
               lin1_W, row(lin1_b), m_W0, row(m_b0), m_W1, row(m_b1),
               m_W2, row(m_b2))


# speedup vs baseline: 27.7067x; 1.0532x over previous
import functools

import jax
import jax.numpy as jnp
from jax import lax
from jax.experimental import pallas as pl
from jax.experimental.pallas import tpu as pltpu
from jax.experimental.pallas import tpu_sc as plsc

N = 8192
B = 8
K = 20
KP = 24
RB = 256
NB = N // RB
CC = 512
E = N * KP
BIG = 1e10
EXBIG = 2e10
IBIG = 2**30

_f32 = jnp.float32


def _dot(a, b):
    return lax.dot_general(a, b, (((1,), (0,)), ((), ())),
                           preferred_element_type=_f32)


def _k1_body(bounds_ref, x0_ref, x0f_ref, x0T_ref, rlo_ref, rhi_ref,
             W0_ref, b0_ref, idxT_ref, A1_ref, B1_ref, key_s):
    r = pl.program_id(0)
    x0b = x0_ref[...]
    W0 = W0_ref[...]
    Wa = W0[0:4, :]
    Wb = W0[4:8, :]
    A1_ref[...] = _dot(x0b, Wa - Wb) + b0_ref[...]
    B1_ref[...] = jnp.concatenate(
        [_dot(x0b, Wb), jnp.zeros((RB, 64), _f32)], axis=1)

    xrT = x0T_ref[...]
    d2r = jnp.sum(xrT * xrT, axis=0, keepdims=True)
    rowg = r * RB + lax.broadcasted_iota(jnp.int32, (1, RB), 1)
    rlo = rlo_ref[...]
    rhi = rhi_ref[...]
    jlo = bounds_ref[r, 0]
    jhi = bounds_ref[r, 1]

    def fill(j, carry):
        coff = pl.multiple_of(j * CC, CC)
        xc = x0f_ref[pl.ds(coff, CC), :]
        dm = _dot(xc, xrT)
        d2c = jnp.sum(xc * xc, axis=1, keepdims=True)
        dist = jnp.maximum(d2c + d2r - 2.0 * dm, 0.0)
        colg = coff + lax.broadcasted_iota(jnp.int32, (CC, 1), 0)
        valid = (colg >= rlo) & (colg < rhi) & (colg != rowg)
        bits = lax.bitcast_convert_type(jnp.where(valid, dist, BIG), jnp.int32)
        key_s[pl.ds(coff, CC), :] = (bits & jnp.int32(-8192)) | colg
        return carry

    lax.fori_loop(jlo, jhi, fill, 0)

    MAXI = 0x7FFFFFFF
    prev = None
    first = None
    for t in range(K):
        def sel(j, macc, prev=prev, t=t):
            coff = pl.multiple_of(j * CC, CC)
            kc = key_s[pl.ds(coff, CC), :]
            if t > 0:
                kc = jnp.where(kc == prev, MAXI, kc)
                key_s[pl.ds(coff, CC), :] = kc
            fold = jnp.min(kc.reshape(CC // 8, 8, RB), axis=0)
            return jnp.minimum(macc, fold)

        macc0 = jnp.full((8, RB), MAXI, jnp.int32)
        macc = lax.fori_loop(jlo, jhi, sel, macc0)
        mk = jnp.min(macc, axis=0, keepdims=True)
        col = mk & 0x1FFF
        idxT_ref[pl.ds(t, 1), :] = col
        prev = mk
        if t == 0:
            first = col
    idxT_ref[pl.ds(K, KP - K), :] = jnp.broadcast_to(first, (KP - K, RB))


def _k1(bounds, x0, x0T, rlo, rhi, W0, b0):
    return pl.pallas_call(
        _k1_body,
        grid=(NB,),
        in_specs=[
            pl.BlockSpec(memory_space=pltpu.SMEM),
            pl.BlockSpec((RB, 4), lambda r: (r, 0)),
            pl.BlockSpec((N, 4), lambda r: (0, 0)),
            pl.BlockSpec((4, RB), lambda r: (0, r)),
            pl.BlockSpec((1, RB), lambda r: (0, r)),
            pl.BlockSpec((1, RB), lambda r: (0, r)),
            pl.BlockSpec((8, 64), lambda r: (0, 0)),
            pl.BlockSpec((1, 64), lambda r: (0, 0)),
        ],
        out_specs=[
            pl.BlockSpec((KP, RB), lambda r: (0, r)),
            pl.BlockSpec((RB, 64), lambda r: (r, 0)),
            pl.BlockSpec((RB, 128), lambda r: (r, 0)),
        ],
        out_shape=[
            jax.ShapeDtypeStruct((KP, N), jnp.int32),
            jax.ShapeDtypeStruct((N, 64), _f32),
            jax.ShapeDtypeStruct((N, 128), _f32),
        ],
        scratch_shapes=[pltpu.VMEM((N, RB), jnp.int32)],
        compiler_params=pltpu.CompilerParams(
            dimension_semantics=("arbitrary",)),
    )(bounds, x0, x0, x0T, rlo, rhi, W0, b0)


def _sc_gather_fn():
    info = plsc.get_sparse_core_info()
    NC, NS = info.num_cores, info.num_subcores
    NW = NC * NS
    per_w = E // NW
    ROWS_W = per_w // 128
    T = 2
    C = T * 128
    steps = per_w // C
    mesh = plsc.VectorSubcoreMesh(core_axis_name="c", subcore_axis_name="s")

    @functools.partial(
        pl.kernel, mesh=mesh,
        out_type=jax.ShapeDtypeStruct((E, 128), _f32),
        scratch_types=[
            pltpu.VMEM((ROWS_W, 128), jnp.int32),
            pltpu.VMEM((C, 128), _f32),
            pltpu.VMEM((C, 128), _f32),
            pltpu.SemaphoreType.DMA,
            pltpu.SemaphoreType.DMA,
            pltpu.SemaphoreType.DMA,
            pltpu.SemaphoreType.DMA,
        ],
    )
    def k(table, idx2, out, idx_v, r0, r1, g0, g1, w0, w1):
        wid = lax.axis_index("s") * NC + lax.axis_index("c")
        pltpu.sync_copy(idx2.at[pl.ds(wid * ROWS_W, ROWS_W)], idx_v)
        base = wid * per_w
        bufs, gsems, wsems = (r0, r1), (g0, g1), (w0, w1)

        def fire(s, buf, sem):
            return [pltpu.async_copy(table.at[idx_v.at[s * T + j]],
                                     buf.at[pl.ds(j * 128, 128)], sem)
                    for j in range(T)]

        gh = fire(0, r0, g0)
        wh = [None, None]
        for s in range(steps):
            cur = s & 1
            for h in gh:
                h.wait()
            wh[cur] = pltpu.async_copy(bufs[cur],
                                       out.at[pl.ds(base + s * C, C)],
                                       wsems[cur])
            if s + 1 < steps:
                nxt = 1 - cur
                if wh[nxt] is not None:
                    wh[nxt].wait()
                gh = fire(s + 1, bufs[nxt], gsems[nxt])
        wh[0].wait()
        wh[1].wait()

    return k


def _sc_gather_max_fn():
    info = plsc.get_sparse_core_info()
    NC, NS = info.num_cores, info.num_subcores
    NW = NC * NS
    rows_w = N // NW
    R = 16
    G = R * KP
    D = 128
    GT = G // 128
    NCH = rows_w // R
    mesh = plsc.VectorSubcoreMesh(core_axis_name="c", subcore_axis_name="s")

    @functools.partial(
        pl.kernel, mesh=mesh,
        out_type=jax.ShapeDtypeStruct((N, D), _f32),
        scratch_types=[
            pltpu.VMEM((rows_w * KP // 128, 128), jnp.int32),
            pltpu.VMEM((G, D), _f32),
            pltpu.VMEM((G, D), _f32),
            pltpu.VMEM((R, D), _f32),
            pltpu.SemaphoreType.DMA,
            pltpu.SemaphoreType.DMA,
        ],
    )
    def k(table, idx2, out, idx_v, b0, b1, out_v, s0, s1):
        wid = lax.axis_index("s") * NC + lax.axis_index("c")
        nrows = rows_w * KP // 128
        pltpu.sync_copy(idx2.at[pl.ds(wid * nrows, nrows)], idx_v)
        bufs, sems = (b0, b1), (s0, s1)

        def fire(c, buf, sem):
            return [pltpu.async_copy(table.at[idx_v.at[c * GT + j]],
                                     buf.at[pl.ds(j * 128, 128)], sem)
                    for j in range(GT)]

        gh = fire(0, b0, s0)
        for c in range(NCH):
            cur = c & 1
            buf = bufs[cur]
            for h in gh:
                h.wait()
            if c + 1 < NCH:
                gh = fire(c + 1, bufs[1 - cur], sems[1 - cur])

            def red(rr, carry, buf=buf):
                rbase = rr * KP
                for q in range(D // 16):
                    a = buf[rbase, pl.ds(q * 16, 16)]
                    for t in range(1, K):
                        a = jnp.maximum(a, buf[rbase + t, pl.ds(q * 16, 16)])
                    out_v[rr, pl.ds(q * 16, 16)] = a
                return carry

            lax.fori_loop(0, R, red, 0)
            obase = pl.multiple_of(wid * rows_w + c * R, 8)
            pltpu.sync_copy(out_v, out.at[pl.ds(obase, R)])

    return k


def _k3_body(G1_ref, A1_ref, g0_ref, be0_ref, W1_ref, b1_ref, g1_ref, be1_ref,
             W2_ref, b2_ref, g2_ref, be2_ref, c2W_ref, c2b_ref,
             x1_ref, A2_ref, B2_ref):
    g1r = G1_ref[:, 0:64]
    a1 = A1_ref[...]
    pre = g1r.reshape(RB, KP, 64) + a1.reshape(RB, 1, 64)
    h = jnp.maximum(pre.reshape(RB * KP, 64), 0.0)
    h = g0_ref[...] * h + be0_ref[...]
    h = jnp.maximum(_dot(h, W1_ref[...]) + b1_ref[...], 0.0)
    h = g1_ref[...] * h + be1_ref[...]
    h = jnp.maximum(_dot(h, W2_ref[...]) + b2_ref[...], 0.0)
    h = g2_ref[...] * h + be2_ref[...]
    x1 = jnp.max(h.reshape(RB, KP, 64), axis=1)
    x1_ref[...] = x1
    c2W = c2W_ref[...]
    Wa = c2W[0:64, :]
    Wb = c2W[64:128, :]
    A2_ref[...] = _dot(x1, Wa - Wb) + c2b_ref[...]
    B2_ref[...] = _dot(x1, Wb)


def _k3(G1, A1, g0, be0, W1, b1, g1, be1, W2, b2, g2, be2, c2W, c2b):
    full64 = pl.BlockSpec((1, 64), lambda r: (0, 0))
    return pl.pallas_call(
        _k3_body,
        grid=(NB,),
        in_specs=[
            pl.BlockSpec((RB * KP, 128), lambda r: (r, 0)),
            pl.BlockSpec((RB, 64), lambda r: (r, 0)),
            full64, full64,
            pl.BlockSpec((64, 64), lambda r: (0, 0)), full64,
            full64, full64,
            pl.BlockSpec((64, 64), lambda r: (0, 0)), full64,
            full64, full64,
            pl.BlockSpec((128, 128), lambda r: (0, 0)),
            pl.BlockSpec((1, 128), lambda r: (0, 0)),
        ],
        out_specs=[
            pl.BlockSpec((RB, 64), lambda r: (r, 0)),
            pl.BlockSpec((RB, 128), lambda r: (r, 0)),
            pl.BlockSpec((RB, 128), lambda r: (r, 0)),
        ],
        out_shape=[
            jax.ShapeDtypeStruct((N, 64), _f32),
            jax.ShapeDtypeStruct((N, 128), _f32),
            jax.ShapeDtypeStruct((N, 128), _f32),
        ],
        compiler_params=pltpu.CompilerParams(
            dimension_semantics=("arbitrary",)),
    )(G1, A1, g0, be0, W1, b1, g1, be1, W2, b2, g2, be2, c2W, c2b)


def _k5_body(x1_ref, A2_ref, M2_ref, bcol_ref, c2g_ref, c2be_ref,
             lW_ref, lb_ref, mW0_ref, mb0_ref, mW1_ref, mb1_ref,
             mW2_ref, mb2_ref, out_ref, acc_ref):
    r = pl.program_id(0)

    @pl.when(r == 0)
    def _init():
        acc_ref[...] = jnp.full((B, 1024), -jnp.inf, _f32)

    x2 = c2g_ref[...] * jnp.maximum(A2_ref[...] + M2_ref[...], 0.0) \
        + c2be_ref[...]
    lW = lW_ref[...]
    o1 = _dot(x1_ref[...], lW[0:64, :]) + _dot(x2, lW[64:192, :]) + lb_ref[...]
    bcol = bcol_ref[...]
    for c in range(B):
        m = jnp.where(bcol == c, o1, -jnp.inf)
        mc = jnp.max(m, axis=0, keepdims=True)
        acc_ref[pl.ds(c, 1), :] = jnp.maximum(acc_ref[pl.ds(c, 1), :], mc)

    @pl.when(r == NB - 1)
    def _head():
        o2 = acc_ref[...]
        h = jnp.maximum(_dot(o2, mW0_ref[...]) + mb0_ref[...], 0.0)
        h = jnp.maximum(_dot(h, mW1_ref[...]) + mb1_ref[...], 0.0)
        o = _dot(h, mW2_ref[...]) + mb2_ref[...]
        mx = jnp.max(o, axis=1, keepdims=True)
        lse = jnp.log(jnp.sum(jnp.exp(o - mx), axis=1, keepdims=True))
        out_ref[...] = o - mx - lse


def _k5(x1, A2, M2, bcol, c2g, c2be, lW, lb, mW0, mb0, mW1, mb1, mW2, mb2):
    def full(shape):
        return pl.BlockSpec(shape, lambda r: (0, 0))
    return pl.pallas_call(
        _k5_body,
        grid=(NB,),
        in_specs=[
            pl.BlockSpec((RB, 64), lambda r: (r, 0)),
            pl.BlockSpec((RB, 128), lambda r: (r, 0)),
            pl.BlockSpec((RB, 128), lambda r: (r, 0)),
            pl.BlockSpec((RB, 1), lambda r: (r, 0)),
            full((1, 128)), full((1, 128)),
            full((192, 1024)), full((1, 1024)),
            full((1024, 512)), full((1, 512)),
            full((512, 256)), full((1, 256)),
            full((256, 40)), full((1, 40)),
        ],
        out_specs=pl.BlockSpec((B, 40), lambda r: (0, 0)),
        out_shape=jax.ShapeDtypeStruct((B, 40), _f32),
        scratch_shapes=[pltpu.VMEM((B, 1024), _f32)],
        compiler_params=pltpu.CompilerParams(
            dimension_semantics=("arbitrary",)),
    )(x1, A2, M2, bcol, c2g, c2be, lW, lb, mW0, mb0, mW1, mb1, mW2, mb2)


def kernel(pos, x, batch, c1_W0, c1_b0, c1_g0, c1_be0, c1_W1, c1_b1, c1_g1,
           c1_be1, c1_W2, c1_b2, c1_g2, c1_be2, c2_W, c2_b, c2_g, c2_be,
           lin1_W, lin1_b, m_W0, m_b0, m_W1, m_b1, m_W2, m_b2):
    x0 = jnp.concatenate([pos, x], axis=1)
    x0T = x0.T
    batch = batch.astype(jnp.int32)
    starts = jnp.searchsorted(batch, jnp.arange(B + 1, dtype=jnp.int32),
                              side="left").astype(jnp.int32)
    rb = jnp.arange(NB, dtype=jnp.int32) * RB
    cf = batch[rb]
    cl = batch[rb + RB - 1]
    jlo = starts[cf] // CC
    jhi = (starts[cl + 1] + CC - 1) // CC
    bounds = jnp.stack([jlo, jhi], axis=1).astype(jnp.int32)

    row = lambda v: v.reshape(1, -1)
    rlo = starts[batch].reshape(1, N)
    rhi = starts[batch + 1].reshape(1, N)
    idxT, A1, B1 = _k1(bounds, x0, x0T, rlo, rhi, c1_W0, row(c1_b0))
    idx2 = idxT.T.reshape(E // 128, 128)
    G1 = _sc_gather_fn()(B1, idx2)
    x1, A2, B2 = _k3(G1, A1, row(c1_g0), row(c1_be0), c1_W1, row(c1_b1),
                     row(c1_g1), row(c1_be1), c1_W2, row(c1_b2),
                     row(c1_g2), row(c1_be2), c2_W, row(c2_b))
    M2 = _sc_gather_max_fn()(B2, idx2)
    return _k5(x1, A2, M2, batch.reshape(N, 1), row(c2_g), row(c2_be),
               lin1_W, row(lin1_b), m_W0, row(m_b0), m_W1, row(m_b1),
               m_W2, row(m_b2))
